# Initial kernel scaffold; baseline (speedup 1.0000x reference)
import jax
import jax.numpy as jnp
from jax.experimental import pallas as pl


def kernel(x, adj_indices, adj_values, Wslice, bslice, Wq, bq, Wk, bk, Wv, bv, Wo, bo, beta_raw):
    raise NotImplementedError("write your pallas kernel here")


# speedup vs baseline: 2.6518x; 2.6518x over previous
import functools
import math

import jax
import jax.numpy as jnp
from jax import lax
from jax.experimental import pallas as pl
from jax.experimental.pallas import tpu as pltpu
from jax.experimental.pallas import tpu_sc as plsc

N = 10000
C = 256
S = 64
H = 8
D = C // H
E = 160000
EPSK = 1e-06

NCHUNK = 1000
ECHUNK = 2000
IDXW = 125
ROWS_PER_ITER = 1000
NW = 32
PER_W = 2 * E // NW
SP = 128

_HI = jax.lax.Precision.HIGHEST


def _k1_body(x_ref, wst_ref, bs_ref, w_ref, sl_ref, ws_ref):
    i = pl.program_id(0)
    xb = x_ref[...]
    logits = jnp.dot(xb, wst_ref[...], preferred_element_type=jnp.float32,
                     precision=_HI) + bs_ref[...]
    m = jnp.max(logits, axis=1, keepdims=True)
    e = jnp.exp(logits - m)
    w = e / jnp.sum(e, axis=1, keepdims=True)
    w_ref[...] = jnp.concatenate(
        [w, jnp.zeros((w.shape[0], SP - S), jnp.float32)], axis=1)

    @pl.when(i == 0)
    def _():
        sl_ref[...] = jnp.zeros_like(sl_ref)
        ws_ref[...] = jnp.zeros_like(ws_ref)

    sl_ref[...] += lax.dot_general(w, xb, (((0,), (0,)), ((), ())),
                                   preferred_element_type=jnp.float32,
                                   precision=_HI)
    ws_ref[...] += jnp.sum(w, axis=0, keepdims=True)


def _k1(x2, wst, bs):
    return pl.pallas_call(
        _k1_body,
        grid=(N // NCHUNK,),
        in_specs=[
            pl.BlockSpec((NCHUNK, C), lambda i: (i, 0)),
            pl.BlockSpec((C, S), lambda i: (0, 0)),
            pl.BlockSpec((1, S), lambda i: (0, 0)),
        ],
        out_specs=[
            pl.BlockSpec((NCHUNK, SP), lambda i: (i, 0)),
            pl.BlockSpec((S, C), lambda i: (0, 0)),
            pl.BlockSpec((1, S), lambda i: (0, 0)),
        ],
        out_shape=[
            jax.ShapeDtypeStruct((N, SP), jnp.float32),
            jax.ShapeDtypeStruct((S, C), jnp.float32),
            jax.ShapeDtypeStruct((1, S), jnp.float32),
        ],
    )(x2, wst, bs)


def _gather_rows(weights, idx2):
    mesh = plsc.VectorSubcoreMesh(core_axis_name="c", subcore_axis_name="s")

    @functools.partial(
        pl.kernel,
        mesh=mesh,
        out_type=jax.ShapeDtypeStruct((2 * E, SP), jnp.float32),
        scratch_types=[
            pltpu.VMEM((8, IDXW), jnp.int32),
            pltpu.VMEM((ROWS_PER_ITER, SP), jnp.float32),
            pltpu.SemaphoreType.DMA,
        ],
    )
    def k(w_hbm, idx_hbm, out_hbm, idx_v, rows_v, sem):
        cid = lax.axis_index("c")
        sid = lax.axis_index("s")
        wid = sid * 2 + cid

        def body(i, carry):
            row0 = wid * (PER_W // IDXW) + i * 8
            pltpu.sync_copy(idx_hbm.at[pl.ds(row0, 8)], idx_v)
            cps = [
                pltpu.async_copy(w_hbm.at[idx_v.at[j]],
                                 rows_v.at[pl.ds(j * IDXW, IDXW)], sem)
                for j in range(8)
            ]
            for cp in cps:
                cp.wait()
            pltpu.sync_copy(
                rows_v, out_hbm.at[pl.ds(wid * PER_W + i * ROWS_PER_ITER,
                                         ROWS_PER_ITER)])
            return carry

        lax.fori_loop(0, PER_W // ROWS_PER_ITER, body, 0)

    return k(weights, idx2)


def _k3_body(wd_ref, ws_ref, v_ref, b_ref):
    i = pl.program_id(0)

    @pl.when(i == 0)
    def _():
        b_ref[...] = jnp.zeros_like(b_ref)

    scaled = wd_ref[...] * v_ref[...]
    b_ref[...] += lax.dot_general(scaled, ws_ref[...], (((0,), (0,)), ((), ())),
                                  preferred_element_type=jnp.float32,
                                  precision=_HI)


def _k3(gath, vals2):
    nblk = E // ECHUNK
    return pl.pallas_call(
        _k3_body,
        grid=(nblk,),
        in_specs=[
            pl.BlockSpec((ECHUNK, SP), lambda i: (i, 0)),
            pl.BlockSpec((ECHUNK, SP), lambda i: (i + E // ECHUNK, 0)),
            pl.BlockSpec((ECHUNK, 1), lambda i: (i, 0)),
        ],
        out_specs=pl.BlockSpec((SP, SP), lambda i: (0, 0)),
        out_shape=jax.ShapeDtypeStruct((SP, SP), jnp.float32),
    )(gath, gath, vals2)


def _k5_body(w_ref, sl_ref, ws_ref, b_ref, wqt_ref, bq_ref, wkt_ref, bk_ref,
             wvt_ref, bv_ref, wot_ref, bo_ref, beta_ref, out_ref, so_ref):
    i = pl.program_id(0)

    @pl.when(i == 0)
    def _():
        rows = lax.broadcasted_iota(jnp.int32, (S, S), 0)
        cols = lax.broadcasted_iota(jnp.int32, (S, S), 1)
        eyemask = rows == cols
        eyef = jnp.where(eyemask, 1.0, 0.0).astype(jnp.float32)

        recip = 1.0 / jnp.maximum(ws_ref[...], EPSK)
        diagm = jnp.where(eyemask, jnp.broadcast_to(recip, (S, S)), 0.0)
        slices = jnp.dot(diagm, sl_ref[...],
                         preferred_element_type=jnp.float32, precision=_HI)

        braw = b_ref[...][:S, :S]
        brawT = lax.dot_general(braw, eyef, (((0,), (0,)), ((), ())),
                                preferred_element_type=jnp.float32,
                                precision=_HI)
        gb = 0.5 * (braw + brawT)
        gb = gb / jnp.maximum(jnp.sum(gb, axis=1, keepdims=True), EPSK)
        gb = jnp.log(jnp.maximum(gb, EPSK))
        br = beta_ref[0, 0]
        beta = jnp.maximum(br, 0.0) + jnp.log1p(jnp.exp(-jnp.abs(br)))
        bias = beta * gb

        q = jnp.dot(slices, wqt_ref[...], preferred_element_type=jnp.float32,
                    precision=_HI) + bq_ref[...]
        kk = jnp.dot(slices, wkt_ref[...], preferred_element_type=jnp.float32,
                     precision=_HI) + bk_ref[...]
        vv = jnp.dot(slices, wvt_ref[...], preferred_element_type=jnp.float32,
                     precision=_HI) + bv_ref[...]

        heads = []
        scale = 1.0 / math.sqrt(D)
        for h in range(H):
            qh = q[:, h * D:(h + 1) * D]
            kh = kk[:, h * D:(h + 1) * D]
            vh = vv[:, h * D:(h + 1) * D]
            logits = lax.dot_general(qh, kh, (((1,), (1,)), ((), ())),
                                     preferred_element_type=jnp.float32,
                                     precision=_HI) * scale + bias
            m = jnp.max(logits, axis=1, keepdims=True)
            p = jnp.exp(logits - m)
            attn = p / jnp.sum(p, axis=1, keepdims=True)
            heads.append(jnp.dot(attn, vh, preferred_element_type=jnp.float32,
                                 precision=_HI))
        cat = jnp.concatenate(heads, axis=1)
        so = jnp.dot(cat, wot_ref[...], preferred_element_type=jnp.float32,
                     precision=_HI) + bo_ref[...]
        so_ref[...] = jnp.concatenate(
            [so, jnp.zeros((SP - S, C), jnp.float32)], axis=0)

    out_ref[...] = jnp.dot(w_ref[...], so_ref[...],
                           preferred_element_type=jnp.float32, precision=_HI)


def _k5(weights, slices_raw, wsum, bias_raw, wqt, bq, wkt, bk, wvt, bv,
        wot, bo, beta):
    const = lambda i: (0, 0)
    return pl.pallas_call(
        _k5_body,
        grid=(N // NCHUNK,),
        in_specs=[
            pl.BlockSpec((NCHUNK, SP), lambda i: (i, 0)),
            pl.BlockSpec((S, C), const),
            pl.BlockSpec((1, S), const),
            pl.BlockSpec((SP, SP), const),
            pl.BlockSpec((C, C), const),
            pl.BlockSpec((1, C), const),
            pl.BlockSpec((C, C), const),
            pl.BlockSpec((1, C), const),
            pl.BlockSpec((C, C), const),
            pl.BlockSpec((1, C), const),
            pl.BlockSpec((C, C), const),
            pl.BlockSpec((1, C), const),
            pl.BlockSpec((1, 1), const),
        ],
        out_specs=pl.BlockSpec((NCHUNK, C), lambda i: (i, 0)),
        out_shape=jax.ShapeDtypeStruct((N, C), jnp.float32),
        scratch_shapes=[pltpu.VMEM((SP, C), jnp.float32)],
    )(weights, slices_raw, wsum, bias_raw, wqt, bq, wkt, bk, wvt, bv,
      wot, bo, beta)


def kernel(x, adj_indices, adj_values, Wslice, bslice, Wq, bq, Wk, bk,
           Wv, bv, Wo, bo, beta_raw):
    x2 = x.reshape(N, C)
    weights, slices_raw, wsum = _k1(x2, Wslice.T, bslice.reshape(1, S))
    idx2 = adj_indices.reshape(2 * E // IDXW, IDXW)
    gath = _gather_rows(weights, idx2)
    bias_raw = _k3(gath, adj_values.reshape(E, 1))
    out2 = _k5(weights, slices_raw, wsum, bias_raw,
               Wq.T, bq.reshape(1, C), Wk.T, bk.reshape(1, C),
               Wv.T, bv.reshape(1, C), Wo.T, bo.reshape(1, C),
               beta_raw.reshape(1, 1))
    return out2.reshape(1, N, C)


# speedup vs baseline: 3.0761x; 1.1600x over previous
import functools
import math

import jax
import jax.numpy as jnp
from jax import lax
from jax.experimental import pallas as pl
from jax.experimental.pallas import tpu as pltpu
from jax.experimental.pallas import tpu_sc as plsc

N = 10000
C = 256
S = 64
H = 8
D = C // H
E = 160000
EPSK = 1e-06

NCHUNK = 1000
ECHUNK = 2000
IDXW = 125
ROWS_PER_ITER = 1000
NW = 32
PER_W = 2 * E // NW
SP = 128


def _k1_body(x_ref, wst_ref, bs_ref, w_ref, sl_ref, ws_ref):
    i = pl.program_id(0)
    xb = x_ref[...]
    logits = jnp.dot(xb, wst_ref[...], preferred_element_type=jnp.float32) + bs_ref[...]
    m = jnp.max(logits, axis=1, keepdims=True)
    e = jnp.exp(logits - m)
    w = e / jnp.sum(e, axis=1, keepdims=True)
    w_ref[...] = jnp.concatenate(
        [w, jnp.zeros((w.shape[0], SP - S), jnp.float32)], axis=1)

    @pl.when(i == 0)
    def _():
        sl_ref[...] = jnp.zeros_like(sl_ref)
        ws_ref[...] = jnp.zeros_like(ws_ref)

    sl_ref[...] += lax.dot_general(w, xb, (((0,), (0,)), ((), ())),
                                   preferred_element_type=jnp.float32)
    ws_ref[...] += jnp.sum(w, axis=0, keepdims=True)


def _k1(x2, wst, bs):
    return pl.pallas_call(
        _k1_body,
        grid=(N // NCHUNK,),
        in_specs=[
            pl.BlockSpec((NCHUNK, C), lambda i: (i, 0)),
            pl.BlockSpec((C, S), lambda i: (0, 0)),
            pl.BlockSpec((1, S), lambda i: (0, 0)),
        ],
        out_specs=[
            pl.BlockSpec((NCHUNK, SP), lambda i: (i, 0)),
            pl.BlockSpec((S, C), lambda i: (0, 0)),
            pl.BlockSpec((1, S), lambda i: (0, 0)),
        ],
        out_shape=[
            jax.ShapeDtypeStruct((N, SP), jnp.float32),
            jax.ShapeDtypeStruct((S, C), jnp.float32),
            jax.ShapeDtypeStruct((1, S), jnp.float32),
        ],
    )(x2, wst, bs)


def _gather_rows(weights, idx2):
    mesh = plsc.VectorSubcoreMesh(core_axis_name="c", subcore_axis_name="s")

    @functools.partial(
        pl.kernel,
        mesh=mesh,
        out_type=jax.ShapeDtypeStruct((2 * E, SP), jnp.float32),
        scratch_types=[
            pltpu.VMEM((8, IDXW), jnp.int32),
            pltpu.VMEM((ROWS_PER_ITER, SP), jnp.float32),
            pltpu.SemaphoreType.DMA,
        ],
    )
    def k(w_hbm, idx_hbm, out_hbm, idx_v, rows_v, sem):
        cid = lax.axis_index("c")
        sid = lax.axis_index("s")
        wid = sid * 2 + cid

        def body(i, carry):
            row0 = wid * (PER_W // IDXW) + i * 8
            pltpu.sync_copy(idx_hbm.at[pl.ds(row0, 8)], idx_v)
            cps = [
                pltpu.async_copy(w_hbm.at[idx_v.at[j]],
                                 rows_v.at[pl.ds(j * IDXW, IDXW)], sem)
                for j in range(8)
            ]
            for cp in cps:
                cp.wait()
            pltpu.sync_copy(
                rows_v, out_hbm.at[pl.ds(wid * PER_W + i * ROWS_PER_ITER,
                                         ROWS_PER_ITER)])
            return carry

        lax.fori_loop(0, PER_W // ROWS_PER_ITER, body, 0)

    return k(weights, idx2)


def _k3_body(wd_ref, ws_ref, v_ref, b_ref):
    i = pl.program_id(0)

    @pl.when(i == 0)
    def _():
        b_ref[...] = jnp.zeros_like(b_ref)

    scaled = wd_ref[...][:, :S] * v_ref[...]
    b_ref[...] += lax.dot_general(scaled, ws_ref[...][:, :S],
                                  (((0,), (0,)), ((), ())),
                                  preferred_element_type=jnp.float32)


def _k3(gath, vals2):
    nblk = E // ECHUNK
    return pl.pallas_call(
        _k3_body,
        grid=(nblk,),
        in_specs=[
            pl.BlockSpec((ECHUNK, SP), lambda i: (i, 0)),
            pl.BlockSpec((ECHUNK, SP), lambda i: (i + E // ECHUNK, 0)),
            pl.BlockSpec((ECHUNK, 1), lambda i: (i, 0)),
        ],
        out_specs=pl.BlockSpec((S, S), lambda i: (0, 0)),
        out_shape=jax.ShapeDtypeStruct((S, S), jnp.float32),
    )(gath, gath, vals2)


def _k5_body(w_ref, sl_ref, ws_ref, b_ref, wqt_ref, bq_ref, wkt_ref, bk_ref,
             wvt_ref, bv_ref, wot_ref, bo_ref, beta_ref, out_ref, so_ref):
    i = pl.program_id(0)

    @pl.when(i == 0)
    def _():
        rows = lax.broadcasted_iota(jnp.int32, (S, S), 0)
        cols = lax.broadcasted_iota(jnp.int32, (S, S), 1)
        eyemask = rows == cols
        eyef = jnp.where(eyemask, 1.0, 0.0).astype(jnp.float32)

        recip = 1.0 / jnp.maximum(ws_ref[...], EPSK)
        diagm = jnp.where(eyemask, jnp.broadcast_to(recip, (S, S)), 0.0)
        slices = jnp.dot(diagm, sl_ref[...],
                         preferred_element_type=jnp.float32)

        braw = b_ref[...][:S, :S]
        brawT = lax.dot_general(braw, eyef, (((0,), (0,)), ((), ())),
                                preferred_element_type=jnp.float32)
        gb = 0.5 * (braw + brawT)
        gb = gb / jnp.maximum(jnp.sum(gb, axis=1, keepdims=True), EPSK)
        gb = jnp.log(jnp.maximum(gb, EPSK))
        br = beta_ref[0, 0]
        beta = jnp.maximum(br, 0.0) + jnp.log1p(jnp.exp(-jnp.abs(br)))
        bias = beta * gb

        q = jnp.dot(slices, wqt_ref[...], preferred_element_type=jnp.float32) + bq_ref[...]
        kk = jnp.dot(slices, wkt_ref[...], preferred_element_type=jnp.float32) + bk_ref[...]
        vv = jnp.dot(slices, wvt_ref[...], preferred_element_type=jnp.float32) + bv_ref[...]

        heads = []
        scale = 1.0 / math.sqrt(D)
        for h in range(H):
            qh = q[:, h * D:(h + 1) * D]
            kh = kk[:, h * D:(h + 1) * D]
            vh = vv[:, h * D:(h + 1) * D]
            logits = lax.dot_general(qh, kh, (((1,), (1,)), ((), ())),
                                     preferred_element_type=jnp.float32) * scale + bias
            m = jnp.max(logits, axis=1, keepdims=True)
            p = jnp.exp(logits - m)
            attn = p / jnp.sum(p, axis=1, keepdims=True)
            heads.append(jnp.dot(attn, vh, preferred_element_type=jnp.float32))
        cat = jnp.concatenate(heads, axis=1)
        so = jnp.dot(cat, wot_ref[...], preferred_element_type=jnp.float32) + bo_ref[...]
        so_ref[...] = jnp.concatenate(
            [so, jnp.zeros((SP - S, C), jnp.float32)], axis=0)

    out_ref[...] = jnp.dot(w_ref[...], so_ref[...],
                           preferred_element_type=jnp.float32)


def _k5(weights, slices_raw, wsum, bias_raw, wqt, bq, wkt, bk, wvt, bv,
        wot, bo, beta):
    const = lambda i: (0, 0)
    return pl.pallas_call(
        _k5_body,
        grid=(N // NCHUNK,),
        in_specs=[
            pl.BlockSpec((NCHUNK, SP), lambda i: (i, 0)),
            pl.BlockSpec((S, C), const),
            pl.BlockSpec((1, S), const),
            pl.BlockSpec((S, S), const),
            pl.BlockSpec((C, C), const),
            pl.BlockSpec((1, C), const),
            pl.BlockSpec((C, C), const),
            pl.BlockSpec((1, C), const),
            pl.BlockSpec((C, C), const),
            pl.BlockSpec((1, C), const),
            pl.BlockSpec((C, C), const),
            pl.BlockSpec((1, C), const),
            pl.BlockSpec((1, 1), const),
        ],
        out_specs=pl.BlockSpec((NCHUNK, C), lambda i: (i, 0)),
        out_shape=jax.ShapeDtypeStruct((N, C), jnp.float32),
        scratch_shapes=[pltpu.VMEM((SP, C), jnp.float32)],
    )(weights, slices_raw, wsum, bias_raw, wqt, bq, wkt, bk, wvt, bv,
      wot, bo, beta)


def kernel(x, adj_indices, adj_values, Wslice, bslice, Wq, bq, Wk, bk,
           Wv, bv, Wo, bo, beta_raw):
    x2 = x.reshape(N, C)
    weights, slices_raw, wsum = _k1(x2, Wslice.T, bslice.reshape(1, S))
    idx2 = adj_indices.reshape(2 * E // IDXW, IDXW)
    gath = _gather_rows(weights, idx2)
    bias_raw = _k3(gath, adj_values.reshape(E, 1))
    out2 = _k5(weights, slices_raw, wsum, bias_raw,
               Wq.T, bq.reshape(1, C), Wk.T, bk.reshape(1, C),
               Wv.T, bv.reshape(1, C), Wo.T, bo.reshape(1, C),
               beta_raw.reshape(1, 1))
    return out2.reshape(1, N, C)


# speedup vs baseline: 3.3575x; 1.0915x over previous
import functools
import math

import jax
import jax.numpy as jnp
from jax import lax
from jax.experimental import pallas as pl
from jax.experimental.pallas import tpu as pltpu
from jax.experimental.pallas import tpu_sc as plsc

N = 10000
C = 256
S = 64
H = 8
D = C // H
E = 160000
EPSK = 1e-06

NCHUNK = 1000
ECHUNK = 2000
IDXW = 125
ROWS_PER_ITER = 1000
NW = 32
PER_W = 2 * E // NW
SP = 128


def _k1_body(x_ref, wst_ref, bs_ref, w_ref, sl_ref, ws_ref):
    i = pl.program_id(0)
    xb = x_ref[...]
    logits = jnp.dot(xb, wst_ref[...], preferred_element_type=jnp.float32) + bs_ref[...]
    m = jnp.max(logits, axis=1, keepdims=True)
    e = jnp.exp(logits - m)
    w = e / jnp.sum(e, axis=1, keepdims=True)
    w_ref[...] = jnp.concatenate(
        [w, jnp.zeros((w.shape[0], SP - S), jnp.float32)], axis=1)

    @pl.when(i == 0)
    def _():
        sl_ref[...] = jnp.zeros_like(sl_ref)
        ws_ref[...] = jnp.zeros_like(ws_ref)

    sl_ref[...] += lax.dot_general(w, xb, (((0,), (0,)), ((), ())),
                                   preferred_element_type=jnp.float32)
    ws_ref[...] += jnp.sum(w, axis=0, keepdims=True)


def _k1(x2, wst, bs):
    return pl.pallas_call(
        _k1_body,
        grid=(N // NCHUNK,),
        in_specs=[
            pl.BlockSpec((NCHUNK, C), lambda i: (i, 0)),
            pl.BlockSpec((C, S), lambda i: (0, 0)),
            pl.BlockSpec((1, S), lambda i: (0, 0)),
        ],
        out_specs=[
            pl.BlockSpec((NCHUNK, SP), lambda i: (i, 0)),
            pl.BlockSpec((S, C), lambda i: (0, 0)),
            pl.BlockSpec((1, S), lambda i: (0, 0)),
        ],
        out_shape=[
            jax.ShapeDtypeStruct((N, SP), jnp.float32),
            jax.ShapeDtypeStruct((S, C), jnp.float32),
            jax.ShapeDtypeStruct((1, S), jnp.float32),
        ],
    )(x2, wst, bs)


def _gather_rows(weights, idx2):
    mesh = plsc.VectorSubcoreMesh(core_axis_name="c", subcore_axis_name="s")

    @functools.partial(
        pl.kernel,
        mesh=mesh,
        out_type=jax.ShapeDtypeStruct((2 * E, SP), jnp.float32),
        scratch_types=[
            pltpu.VMEM((8, IDXW), jnp.int32),
            pltpu.VMEM((ROWS_PER_ITER, SP), jnp.float32),
            pltpu.SemaphoreType.DMA,
        ],
    )
    def k(w_hbm, idx_hbm, out_hbm, idx_v, rows_v, sem):
        cid = lax.axis_index("c")
        sid = lax.axis_index("s")
        wid = sid * 2 + cid

        def body(i, carry):
            row0 = wid * (PER_W // IDXW) + i * 8
            pltpu.sync_copy(idx_hbm.at[pl.ds(row0, 8)], idx_v)
            cps = [
                pltpu.async_copy(w_hbm.at[idx_v.at[j]],
                                 rows_v.at[pl.ds(j * IDXW, IDXW)], sem)
                for j in range(8)
            ]
            for cp in cps:
                cp.wait()
            pltpu.sync_copy(
                rows_v, out_hbm.at[pl.ds(wid * PER_W + i * ROWS_PER_ITER,
                                         ROWS_PER_ITER)])
            return carry

        lax.fori_loop(0, PER_W // ROWS_PER_ITER, body, 0)

    return k(weights, idx2)


def _k3_body(wd_ref, ws_ref, v_ref, b_ref):
    i = pl.program_id(0)

    @pl.when(i == 0)
    def _():
        b_ref[...] = jnp.zeros_like(b_ref)

    vrow = v_ref[...].reshape(1, ECHUNK)
    vcol = lax.dot_general(vrow, jnp.ones((1, 1), jnp.float32),
                           (((0,), (0,)), ((), ())),
                           preferred_element_type=jnp.float32)
    scaled = wd_ref[...][:, :S] * vcol
    b_ref[...] += lax.dot_general(scaled, ws_ref[...][:, :S],
                                  (((0,), (0,)), ((), ())),
                                  preferred_element_type=jnp.float32)


def _k3(gath, vals2):
    nblk = E // ECHUNK
    return pl.pallas_call(
        _k3_body,
        grid=(nblk,),
        in_specs=[
            pl.BlockSpec((ECHUNK, SP), lambda i: (i, 0)),
            pl.BlockSpec((ECHUNK, SP), lambda i: (i + E // ECHUNK, 0)),
            pl.BlockSpec((1, 1, ECHUNK), lambda i: (i, 0, 0)),
        ],
        out_specs=pl.BlockSpec((S, S), lambda i: (0, 0)),
        out_shape=jax.ShapeDtypeStruct((S, S), jnp.float32),
    )(gath, gath, vals2)


def _k5_body(w_ref, sl_ref, ws_ref, b_ref, wqt_ref, bq_ref, wkt_ref, bk_ref,
             wvt_ref, bv_ref, wot_ref, bo_ref, beta_ref, out_ref, so_ref):
    i = pl.program_id(0)

    @pl.when(i == 0)
    def _():
        rows = lax.broadcasted_iota(jnp.int32, (S, S), 0)
        cols = lax.broadcasted_iota(jnp.int32, (S, S), 1)
        eyemask = rows == cols
        eyef = jnp.where(eyemask, 1.0, 0.0).astype(jnp.float32)

        recip = 1.0 / jnp.maximum(ws_ref[...], EPSK)
        diagm = jnp.where(eyemask, jnp.broadcast_to(recip, (S, S)), 0.0)
        slices = jnp.dot(diagm, sl_ref[...],
                         preferred_element_type=jnp.float32)

        braw = b_ref[...][:S, :S]
        brawT = lax.dot_general(braw, eyef, (((0,), (0,)), ((), ())),
                                preferred_element_type=jnp.float32)
        gb = 0.5 * (braw + brawT)
        gb = gb / jnp.maximum(jnp.sum(gb, axis=1, keepdims=True), EPSK)
        gb = jnp.log(jnp.maximum(gb, EPSK))
        br = beta_ref[0, 0]
        beta = jnp.maximum(br, 0.0) + jnp.log1p(jnp.exp(-jnp.abs(br)))
        bias = beta * gb

        q = jnp.dot(slices, wqt_ref[...], preferred_element_type=jnp.float32) + bq_ref[...]
        kk = jnp.dot(slices, wkt_ref[...], preferred_element_type=jnp.float32) + bk_ref[...]
        vv = jnp.dot(slices, wvt_ref[...], preferred_element_type=jnp.float32) + bv_ref[...]

        heads = []
        scale = 1.0 / math.sqrt(D)
        for h in range(H):
            qh = q[:, h * D:(h + 1) * D]
            kh = kk[:, h * D:(h + 1) * D]
            vh = vv[:, h * D:(h + 1) * D]
            logits = lax.dot_general(qh, kh, (((1,), (1,)), ((), ())),
                                     preferred_element_type=jnp.float32) * scale + bias
            m = jnp.max(logits, axis=1, keepdims=True)
            p = jnp.exp(logits - m)
            attn = p / jnp.sum(p, axis=1, keepdims=True)
            heads.append(jnp.dot(attn, vh, preferred_element_type=jnp.float32))
        cat = jnp.concatenate(heads, axis=1)
        so = jnp.dot(cat, wot_ref[...], preferred_element_type=jnp.float32) + bo_ref[...]
        so_ref[...] = jnp.concatenate(
            [so, jnp.zeros((SP - S, C), jnp.float32)], axis=0)

    out_ref[...] = jnp.dot(w_ref[...], so_ref[...],
                           preferred_element_type=jnp.float32)


def _k5(weights, slices_raw, wsum, bias_raw, wqt, bq, wkt, bk, wvt, bv,
        wot, bo, beta):
    const = lambda i: (0, 0)
    return pl.pallas_call(
        _k5_body,
        grid=(N // NCHUNK,),
        in_specs=[
            pl.BlockSpec((NCHUNK, SP), lambda i: (i, 0)),
            pl.BlockSpec((S, C), const),
            pl.BlockSpec((1, S), const),
            pl.BlockSpec((S, S), const),
            pl.BlockSpec((C, C), const),
            pl.BlockSpec((1, C), const),
            pl.BlockSpec((C, C), const),
            pl.BlockSpec((1, C), const),
            pl.BlockSpec((C, C), const),
            pl.BlockSpec((1, C), const),
            pl.BlockSpec((C, C), const),
            pl.BlockSpec((1, C), const),
            pl.BlockSpec((1, 1), const),
        ],
        out_specs=pl.BlockSpec((NCHUNK, C), lambda i: (i, 0)),
        out_shape=jax.ShapeDtypeStruct((N, C), jnp.float32),
        scratch_shapes=[pltpu.VMEM((SP, C), jnp.float32)],
    )(weights, slices_raw, wsum, bias_raw, wqt, bq, wkt, bk, wvt, bv,
      wot, bo, beta)


def kernel(x, adj_indices, adj_values, Wslice, bslice, Wq, bq, Wk, bk,
           Wv, bv, Wo, bo, beta_raw):
    x2 = x.reshape(N, C)
    weights, slices_raw, wsum = _k1(x2, Wslice.T, bslice.reshape(1, S))
    idx2 = adj_indices.reshape(2 * E // IDXW, IDXW)
    gath = _gather_rows(weights, idx2)
    bias_raw = _k3(gath, adj_values.reshape(E // ECHUNK, 1, ECHUNK))
    out2 = _k5(weights, slices_raw, wsum, bias_raw,
               Wq.T, bq.reshape(1, C), Wk.T, bk.reshape(1, C),
               Wv.T, bv.reshape(1, C), Wo.T, bo.reshape(1, C),
               beta_raw.reshape(1, 1))
    return out2.reshape(1, N, C)


# speedup vs baseline: 3.6985x; 1.1016x over previous
import functools
import math

import jax
import jax.numpy as jnp
from jax import lax
from jax.experimental import pallas as pl
from jax.experimental.pallas import tpu as pltpu
from jax.experimental.pallas import tpu_sc as plsc

N = 10000
C = 256
S = 64
H = 8
D = C // H
E = 160000
EPSK = 1e-06

NCHUNK = 1000
ECHUNK = 2000
IDXW = 100
ROWS_PER_ITER = 1000
NW = 32
PER_W = 2 * E // NW
SP = 128


def _k1_body(x_ref, wst_ref, bs_ref, w_ref, sl_ref, ws_ref):
    i = pl.program_id(0)
    xb = x_ref[...]
    logits = jnp.dot(xb, wst_ref[...], preferred_element_type=jnp.float32) + bs_ref[...]
    m = jnp.max(logits, axis=1, keepdims=True)
    e = jnp.exp(logits - m)
    w = e / jnp.sum(e, axis=1, keepdims=True)
    w_ref[...] = jnp.concatenate(
        [w, jnp.zeros((w.shape[0], SP - S), jnp.float32)], axis=1)

    @pl.when(i == 0)
    def _():
        sl_ref[...] = jnp.zeros_like(sl_ref)
        ws_ref[...] = jnp.zeros_like(ws_ref)

    sl_ref[...] += lax.dot_general(w, xb, (((0,), (0,)), ((), ())),
                                   preferred_element_type=jnp.float32)
    ws_ref[...] += jnp.sum(w, axis=0, keepdims=True)


def _k1(x2, wst, bs):
    return pl.pallas_call(
        _k1_body,
        grid=(N // NCHUNK,),
        in_specs=[
            pl.BlockSpec((NCHUNK, C), lambda i: (i, 0)),
            pl.BlockSpec((C, S), lambda i: (0, 0)),
            pl.BlockSpec((1, S), lambda i: (0, 0)),
        ],
        out_specs=[
            pl.BlockSpec((NCHUNK, SP), lambda i: (i, 0)),
            pl.BlockSpec((S, C), lambda i: (0, 0)),
            pl.BlockSpec((1, S), lambda i: (0, 0)),
        ],
        out_shape=[
            jax.ShapeDtypeStruct((N, SP), jnp.float32),
            jax.ShapeDtypeStruct((S, C), jnp.float32),
            jax.ShapeDtypeStruct((1, S), jnp.float32),
        ],
    )(x2, wst, bs)


def _gather_rows(weights, idx2):
    nw, nr, _ = idx2.shape
    per_w = nr * IDXW
    nrows = nw * per_w
    chunk = 2 * IDXW
    niter = per_w // chunk
    mesh = plsc.VectorSubcoreMesh(core_axis_name="c", subcore_axis_name="s")

    @functools.partial(
        pl.kernel,
        mesh=mesh,
        out_type=jax.ShapeDtypeStruct((nrows, SP), jnp.float32),
        scratch_types=[
            pltpu.VMEM((nr, IDXW), jnp.int32),
            pltpu.VMEM((2, 2 * IDXW, SP), jnp.float32),
            pltpu.SemaphoreType.DMA,
            pltpu.SemaphoreType.DMA,
            pltpu.SemaphoreType.DMA,
            pltpu.SemaphoreType.DMA,
        ],
    )
    def k(w_hbm, idx_hbm, out_hbm, idx_v, rows_v, g0, g1, w0, w1):
        cid = lax.axis_index("c")
        sid = lax.axis_index("s")
        wid = sid * 2 + cid
        gsem = [g0, g1]
        wsem = [w0, w1]
        out0 = wid * per_w
        pltpu.sync_copy(idx_hbm.at[wid], idx_v)

        glist = [None, None]
        wr = [None, None]
        for i in range(niter):
            b = i & 1
            if wr[b] is not None:
                wr[b].wait()
                wr[b] = None
            glist[b] = [
                pltpu.async_copy(w_hbm.at[idx_v.at[2 * i + j]],
                                 rows_v.at[b, pl.ds(j * IDXW, IDXW)], gsem[b])
                for j in range(2)
            ]
            if i >= 1:
                pb = (i - 1) & 1
                for cp in glist[pb]:
                    cp.wait()
                wr[pb] = pltpu.async_copy(
                    rows_v.at[pb],
                    out_hbm.at[pl.ds(out0 + (i - 1) * chunk, chunk)], wsem[pb])
        lb = (niter - 1) & 1
        for cp in glist[lb]:
            cp.wait()
        wr[lb] = pltpu.async_copy(
            rows_v.at[lb],
            out_hbm.at[pl.ds(out0 + (niter - 1) * chunk, chunk)], wsem[lb])
        for b in range(2):
            if wr[b] is not None:
                wr[b].wait()

    return k(weights, idx2)


def _k3_body(wd_ref, ws_ref, v_ref, b_ref):
    i = pl.program_id(0)

    @pl.when(i == 0)
    def _():
        b_ref[...] = jnp.zeros_like(b_ref)

    vrow = v_ref[...].reshape(1, ECHUNK)
    vcol = lax.dot_general(vrow, jnp.ones((1, 1), jnp.float32),
                           (((0,), (0,)), ((), ())),
                           preferred_element_type=jnp.float32)
    scaled = wd_ref[...][:, :S] * vcol
    b_ref[...] += lax.dot_general(scaled, ws_ref[...][:, :S],
                                  (((0,), (0,)), ((), ())),
                                  preferred_element_type=jnp.float32)


def _k3(gath, vals2):
    ne = vals2.shape[0] * ECHUNK
    nblk = ne // ECHUNK
    return pl.pallas_call(
        _k3_body,
        grid=(nblk,),
        in_specs=[
            pl.BlockSpec((ECHUNK, SP), lambda i: (i, 0)),
            pl.BlockSpec((ECHUNK, SP), lambda i, _n=nblk: (i + _n, 0)),
            pl.BlockSpec((1, 1, ECHUNK), lambda i: (i, 0, 0)),
        ],
        out_specs=pl.BlockSpec((S, S), lambda i: (0, 0)),
        out_shape=jax.ShapeDtypeStruct((S, S), jnp.float32),
    )(gath, gath, vals2)


def _k5_body(w_ref, sl_ref, ws_ref, b_ref, b2_ref, wqt_ref, bq_ref, wkt_ref, bk_ref,
             wvt_ref, bv_ref, wot_ref, bo_ref, beta_ref, out_ref, so_ref):
    i = pl.program_id(0)

    @pl.when(i == 0)
    def _():
        rows = lax.broadcasted_iota(jnp.int32, (S, S), 0)
        cols = lax.broadcasted_iota(jnp.int32, (S, S), 1)
        eyemask = rows == cols
        eyef = jnp.where(eyemask, 1.0, 0.0).astype(jnp.float32)

        recip = 1.0 / jnp.maximum(ws_ref[...], EPSK)
        diagm = jnp.where(eyemask, jnp.broadcast_to(recip, (S, S)), 0.0)
        slices = jnp.dot(diagm, sl_ref[...],
                         preferred_element_type=jnp.float32)

        braw = (b_ref[...] + b2_ref[...])[:S, :S]
        brawT = lax.dot_general(braw, eyef, (((0,), (0,)), ((), ())),
                                preferred_element_type=jnp.float32)
        gb = 0.5 * (braw + brawT)
        gb = gb / jnp.maximum(jnp.sum(gb, axis=1, keepdims=True), EPSK)
        gb = jnp.log(jnp.maximum(gb, EPSK))
        br = beta_ref[0, 0]
        beta = jnp.maximum(br, 0.0) + jnp.log1p(jnp.exp(-jnp.abs(br)))
        bias = beta * gb

        q = jnp.dot(slices, wqt_ref[...], preferred_element_type=jnp.float32) + bq_ref[...]
        kk = jnp.dot(slices, wkt_ref[...], preferred_element_type=jnp.float32) + bk_ref[...]
        vv = jnp.dot(slices, wvt_ref[...], preferred_element_type=jnp.float32) + bv_ref[...]

        heads = []
        scale = 1.0 / math.sqrt(D)
        for h in range(H):
            qh = q[:, h * D:(h + 1) * D]
            kh = kk[:, h * D:(h + 1) * D]
            vh = vv[:, h * D:(h + 1) * D]
            logits = lax.dot_general(qh, kh, (((1,), (1,)), ((), ())),
                                     preferred_element_type=jnp.float32) * scale + bias
            m = jnp.max(logits, axis=1, keepdims=True)
            p = jnp.exp(logits - m)
            attn = p / jnp.sum(p, axis=1, keepdims=True)
            heads.append(jnp.dot(attn, vh, preferred_element_type=jnp.float32))
        cat = jnp.concatenate(heads, axis=1)
        so = jnp.dot(cat, wot_ref[...], preferred_element_type=jnp.float32) + bo_ref[...]
        so_ref[...] = jnp.concatenate(
            [so, jnp.zeros((SP - S, C), jnp.float32)], axis=0)

    out_ref[...] = jnp.dot(w_ref[...], so_ref[...],
                           preferred_element_type=jnp.float32)


def _k5(weights, slices_raw, wsum, bias_a, bias_b, wqt, bq, wkt, bk, wvt, bv,
        wot, bo, beta):
    const = lambda i: (0, 0)
    return pl.pallas_call(
        _k5_body,
        grid=(N // NCHUNK,),
        in_specs=[
            pl.BlockSpec((NCHUNK, SP), lambda i: (i, 0)),
            pl.BlockSpec((S, C), const),
            pl.BlockSpec((1, S), const),
            pl.BlockSpec((S, S), const),
            pl.BlockSpec((S, S), const),
            pl.BlockSpec((C, C), const),
            pl.BlockSpec((1, C), const),
            pl.BlockSpec((C, C), const),
            pl.BlockSpec((1, C), const),
            pl.BlockSpec((C, C), const),
            pl.BlockSpec((1, C), const),
            pl.BlockSpec((C, C), const),
            pl.BlockSpec((1, C), const),
            pl.BlockSpec((1, 1), const),
        ],
        out_specs=pl.BlockSpec((NCHUNK, C), lambda i: (i, 0)),
        out_shape=jax.ShapeDtypeStruct((N, C), jnp.float32),
        scratch_shapes=[pltpu.VMEM((SP, C), jnp.float32)],
    )(weights, slices_raw, wsum, bias_a, bias_b, wqt, bq, wkt, bk, wvt, bv,
      wot, bo, beta)


def kernel(x, adj_indices, adj_values, Wslice, bslice, Wq, bq, Wk, bk,
           Wv, bv, Wo, bo, beta_raw):
    x2 = x.reshape(N, C)
    weights, slices_raw, wsum = _k1(x2, Wslice.T, bslice.reshape(1, S))
    eh = E // 2
    idx_a = adj_indices[:, :eh].reshape(NW, 2 * eh // (NW * IDXW), IDXW)
    idx_b = adj_indices[:, eh:].reshape(NW, 2 * eh // (NW * IDXW), IDXW)
    gath_a = _gather_rows(weights, idx_a)
    gath_b = _gather_rows(weights, idx_b)
    bias_a = _k3(gath_a, adj_values[:eh].reshape(eh // ECHUNK, 1, ECHUNK))
    bias_b = _k3(gath_b, adj_values[eh:].reshape(eh // ECHUNK, 1, ECHUNK))
    out2 = _k5(weights, slices_raw, wsum, bias_a, bias_b,
               Wq.T, bq.reshape(1, C), Wk.T, bk.reshape(1, C),
               Wv.T, bv.reshape(1, C), Wo.T, bo.reshape(1, C),
               beta_raw.reshape(1, 1))
    return out2.reshape(1, N, C)


# speedup vs baseline: 4.1631x; 1.1256x over previous
import functools
import math

import jax
import jax.numpy as jnp
from jax import lax
from jax.experimental import pallas as pl
from jax.experimental.pallas import tpu as pltpu
from jax.experimental.pallas import tpu_sc as plsc

N = 10000
C = 256
S = 64
H = 8
D = C // H
E = 160000
EPSK = 1e-06

NCHUNK = 2000
ECHUNK = 4000
IDXW = 100
ROWS_PER_ITER = 1000
NW = 32
PER_W = 2 * E // NW
SP = 128


def _k1_body(x_ref, wst_ref, bs_ref, w_ref, sl_ref, ws_ref):
    i = pl.program_id(0)
    xb = x_ref[...]
    logits = lax.dot_general(xb, wst_ref[...], (((1,), (1,)), ((), ())),
                             preferred_element_type=jnp.float32) + bs_ref[...]
    m = jnp.max(logits, axis=1, keepdims=True)
    e = jnp.exp(logits - m)
    w = e / jnp.sum(e, axis=1, keepdims=True)
    w_ref[...] = jnp.concatenate(
        [w, jnp.zeros((w.shape[0], SP - S), jnp.float32)], axis=1)

    @pl.when(i == 0)
    def _():
        sl_ref[...] = jnp.zeros_like(sl_ref)
        ws_ref[...] = jnp.zeros_like(ws_ref)

    sl_ref[...] += lax.dot_general(w, xb, (((0,), (0,)), ((), ())),
                                   preferred_element_type=jnp.float32)
    ws_ref[...] += jnp.sum(w, axis=0, keepdims=True)


def _k1(x2, wst, bs):
    return pl.pallas_call(
        _k1_body,
        grid=(N // NCHUNK,),
        in_specs=[
            pl.BlockSpec((NCHUNK, C), lambda i: (i, 0)),
            pl.BlockSpec((S, C), lambda i: (0, 0)),
            pl.BlockSpec((1, S), lambda i: (0, 0)),
        ],
        out_specs=[
            pl.BlockSpec((NCHUNK, SP), lambda i: (i, 0)),
            pl.BlockSpec((S, C), lambda i: (0, 0)),
            pl.BlockSpec((1, S), lambda i: (0, 0)),
        ],
        out_shape=[
            jax.ShapeDtypeStruct((N, SP), jnp.float32),
            jax.ShapeDtypeStruct((S, C), jnp.float32),
            jax.ShapeDtypeStruct((1, S), jnp.float32),
        ],
    )(x2, wst, bs)


def _gather_rows(weights, idx2):
    nw, nr, _ = idx2.shape
    per_w = nr * IDXW
    nrows = nw * per_w
    chunk = 2 * IDXW
    niter = per_w // chunk
    mesh = plsc.VectorSubcoreMesh(core_axis_name="c", subcore_axis_name="s")

    @functools.partial(
        pl.kernel,
        mesh=mesh,
        out_type=jax.ShapeDtypeStruct((nrows, SP), jnp.float32),
        scratch_types=[
            pltpu.VMEM((nr, IDXW), jnp.int32),
            pltpu.VMEM((2, 2 * IDXW, SP), jnp.float32),
            pltpu.SemaphoreType.DMA,
            pltpu.SemaphoreType.DMA,
            pltpu.SemaphoreType.DMA,
            pltpu.SemaphoreType.DMA,
        ],
    )
    def k(w_hbm, idx_hbm, out_hbm, idx_v, rows_v, g0, g1, w0, w1):
        cid = lax.axis_index("c")
        sid = lax.axis_index("s")
        wid = sid * 2 + cid
        gsem = [g0, g1]
        wsem = [w0, w1]
        out0 = wid * per_w
        pltpu.sync_copy(idx_hbm.at[wid], idx_v)

        glist = [None, None]
        wr = [None, None]
        for i in range(niter):
            b = i & 1
            if wr[b] is not None:
                wr[b].wait()
                wr[b] = None
            glist[b] = [
                pltpu.async_copy(w_hbm.at[idx_v.at[2 * i + j]],
                                 rows_v.at[b, pl.ds(j * IDXW, IDXW)], gsem[b])
                for j in range(2)
            ]
            if i >= 1:
                pb = (i - 1) & 1
                for cp in glist[pb]:
                    cp.wait()
                wr[pb] = pltpu.async_copy(
                    rows_v.at[pb],
                    out_hbm.at[pl.ds(out0 + (i - 1) * chunk, chunk)], wsem[pb])
        lb = (niter - 1) & 1
        for cp in glist[lb]:
            cp.wait()
        wr[lb] = pltpu.async_copy(
            rows_v.at[lb],
            out_hbm.at[pl.ds(out0 + (niter - 1) * chunk, chunk)], wsem[lb])
        for b in range(2):
            if wr[b] is not None:
                wr[b].wait()

    return k(weights, idx2)


def _k3_body(wd_ref, ws_ref, v_ref, b_ref):
    i = pl.program_id(0)

    @pl.when(i == 0)
    def _():
        b_ref[...] = jnp.zeros_like(b_ref)

    vrow = v_ref[...].reshape(1, ECHUNK)
    vcol = lax.dot_general(vrow, jnp.ones((1, 1), jnp.float32),
                           (((0,), (0,)), ((), ())),
                           preferred_element_type=jnp.float32)
    scaled = wd_ref[...][:, :S] * vcol
    b_ref[...] += lax.dot_general(scaled, ws_ref[...][:, :S],
                                  (((0,), (0,)), ((), ())),
                                  preferred_element_type=jnp.float32)


def _k3(gath, vals2):
    ne = vals2.shape[0] * ECHUNK
    nblk = ne // ECHUNK
    return pl.pallas_call(
        _k3_body,
        grid=(nblk,),
        in_specs=[
            pl.BlockSpec((ECHUNK, SP), lambda i: (i, 0)),
            pl.BlockSpec((ECHUNK, SP), lambda i, _n=nblk: (i + _n, 0)),
            pl.BlockSpec((1, 1, ECHUNK), lambda i: (i, 0, 0)),
        ],
        out_specs=pl.BlockSpec((S, S), lambda i: (0, 0)),
        out_shape=jax.ShapeDtypeStruct((S, S), jnp.float32),
    )(gath, gath, vals2)


def _k5_body(w_ref, sl_ref, ws_ref, b_ref, b2_ref, wqt_ref, bq_ref, wkt_ref, bk_ref,
             wvt_ref, bv_ref, wot_ref, bo_ref, beta_ref, out_ref, so_ref):
    i = pl.program_id(0)

    @pl.when(i == 0)
    def _():
        rows = lax.broadcasted_iota(jnp.int32, (S, S), 0)
        cols = lax.broadcasted_iota(jnp.int32, (S, S), 1)
        eyemask = rows == cols
        eyef = jnp.where(eyemask, 1.0, 0.0).astype(jnp.float32)

        recip = 1.0 / jnp.maximum(ws_ref[...], EPSK)
        diagm = jnp.where(eyemask, jnp.broadcast_to(recip, (S, S)), 0.0)
        slices = jnp.dot(diagm, sl_ref[...],
                         preferred_element_type=jnp.float32)

        braw = (b_ref[...] + b2_ref[...])[:S, :S]
        brawT = lax.dot_general(braw, eyef, (((0,), (0,)), ((), ())),
                                preferred_element_type=jnp.float32)
        gb = 0.5 * (braw + brawT)
        gb = gb / jnp.maximum(jnp.sum(gb, axis=1, keepdims=True), EPSK)
        gb = jnp.log(jnp.maximum(gb, EPSK))
        br = beta_ref[0, 0]
        beta = jnp.maximum(br, 0.0) + jnp.log1p(jnp.exp(-jnp.abs(br)))
        bias = beta * gb

        q = lax.dot_general(slices, wqt_ref[...], (((1,), (1,)), ((), ())),
                            preferred_element_type=jnp.float32) + bq_ref[...]
        kk = lax.dot_general(slices, wkt_ref[...], (((1,), (1,)), ((), ())),
                             preferred_element_type=jnp.float32) + bk_ref[...]
        vv = lax.dot_general(slices, wvt_ref[...], (((1,), (1,)), ((), ())),
                             preferred_element_type=jnp.float32) + bv_ref[...]

        heads = []
        scale = 1.0 / math.sqrt(D)
        for h in range(H):
            qh = q[:, h * D:(h + 1) * D]
            kh = kk[:, h * D:(h + 1) * D]
            vh = vv[:, h * D:(h + 1) * D]
            logits = lax.dot_general(qh, kh, (((1,), (1,)), ((), ())),
                                     preferred_element_type=jnp.float32) * scale + bias
            m = jnp.max(logits, axis=1, keepdims=True)
            p = jnp.exp(logits - m)
            attn = p / jnp.sum(p, axis=1, keepdims=True)
            heads.append(jnp.dot(attn, vh, preferred_element_type=jnp.float32))
        cat = jnp.concatenate(heads, axis=1)
        so = lax.dot_general(cat, wot_ref[...], (((1,), (1,)), ((), ())),
                             preferred_element_type=jnp.float32) + bo_ref[...]
        so_ref[...] = jnp.concatenate(
            [so, jnp.zeros((SP - S, C), jnp.float32)], axis=0)

    out_ref[...] = jnp.dot(w_ref[...], so_ref[...],
                           preferred_element_type=jnp.float32)


def _k5(weights, slices_raw, wsum, bias_a, bias_b, wqt, bq, wkt, bk, wvt, bv,
        wot, bo, beta):
    const = lambda i: (0, 0)
    return pl.pallas_call(
        _k5_body,
        grid=(N // NCHUNK,),
        in_specs=[
            pl.BlockSpec((NCHUNK, SP), lambda i: (i, 0)),
            pl.BlockSpec((S, C), const),
            pl.BlockSpec((1, S), const),
            pl.BlockSpec((S, S), const),
            pl.BlockSpec((S, S), const),
            pl.BlockSpec((C, C), const),
            pl.BlockSpec((1, C), const),
            pl.BlockSpec((C, C), const),
            pl.BlockSpec((1, C), const),
            pl.BlockSpec((C, C), const),
            pl.BlockSpec((1, C), const),
            pl.BlockSpec((C, C), const),
            pl.BlockSpec((1, C), const),
            pl.BlockSpec((1, 1), const),
        ],
        out_specs=pl.BlockSpec((NCHUNK, C), lambda i: (i, 0)),
        out_shape=jax.ShapeDtypeStruct((N, C), jnp.float32),
        scratch_shapes=[pltpu.VMEM((SP, C), jnp.float32)],
    )(weights, slices_raw, wsum, bias_a, bias_b, wqt, bq, wkt, bk, wvt, bv,
      wot, bo, beta)


def kernel(x, adj_indices, adj_values, Wslice, bslice, Wq, bq, Wk, bk,
           Wv, bv, Wo, bo, beta_raw):
    x2 = x.reshape(N, C)
    weights, slices_raw, wsum = _k1(x2, Wslice, bslice.reshape(1, S))
    eh = E // 2
    idx_a = adj_indices[:, :eh].reshape(NW, 2 * eh // (NW * IDXW), IDXW)
    idx_b = adj_indices[:, eh:].reshape(NW, 2 * eh // (NW * IDXW), IDXW)
    gath_a = _gather_rows(weights, idx_a)
    gath_b = _gather_rows(weights, idx_b)
    bias_a = _k3(gath_a, adj_values[:eh].reshape(eh // ECHUNK, 1, ECHUNK))
    bias_b = _k3(gath_b, adj_values[eh:].reshape(eh // ECHUNK, 1, ECHUNK))
    out2 = _k5(weights, slices_raw, wsum, bias_a, bias_b,
               Wq, bq.reshape(1, C), Wk, bk.reshape(1, C),
               Wv, bv.reshape(1, C), Wo, bo.reshape(1, C),
               beta_raw.reshape(1, 1))
    return out2.reshape(1, N, C)
